# baseline (speedup 1.0000x reference)
import functools

import jax
import jax.numpy as jnp
from jax import lax
from jax.experimental import pallas as pl
from jax.experimental.pallas import tpu as pltpu
from jax.experimental.pallas import tpu_sc as plsc

B, S, F = 4, 4096, 2048
L = 16
F_W = 128
S_CH = 256
N_CH = S // S_CH
N_VEC = F_W // L
N_FBLK = F // F_W

_MESH = plsc.VectorSubcoreMesh(core_axis_name="c", subcore_axis_name="s")
NW = _MESH.num_cores * _MESH.num_subcores
UNITS_PER_W = (B * N_FBLK) // NW


@functools.partial(
    pl.kernel,
    mesh=_MESH,
    out_type=jax.ShapeDtypeStruct((B, S, F), jnp.float32),
    scratch_types=[
        pltpu.VMEM((S_CH, F_W), jnp.float32),
        pltpu.VMEM((S_CH, F_W), jnp.float32),
    ],
)
def _sc_excl_cumsum(x_hbm, out_hbm, in_v, out_v):
    wid = lax.axis_index("s") * _MESH.num_cores + lax.axis_index("c")

    def row_body(i, accs):
        in_row = in_v.at[i]
        out_row = out_v.at[i]
        new = []
        for v in range(N_VEC):
            vec = in_row[pl.ds(v * L, L)]
            out_row[pl.ds(v * L, L)] = accs[v]
            new.append(accs[v] + vec)
        return tuple(new)

    for k in range(UNITS_PER_W):
        u = wid * UNITS_PER_W + k
        b = u // N_FBLK
        f0 = pl.multiple_of((u % N_FBLK) * F_W, F_W)

        def chunk_body(ch, accs):
            s0 = pl.multiple_of(ch * S_CH, S_CH)
            pltpu.sync_copy(x_hbm.at[b, pl.ds(s0, S_CH), pl.ds(f0, F_W)],
                            in_v)
            accs = lax.fori_loop(0, S_CH, row_body, accs, unroll=8)
            pltpu.sync_copy(out_v,
                            out_hbm.at[b, pl.ds(s0, S_CH), pl.ds(f0, F_W)])
            return accs

        zero = jnp.zeros((L,), jnp.float32)
        lax.fori_loop(0, N_CH, chunk_body, (zero,) * N_VEC)


def kernel(x):
    return _sc_excl_cumsum(x)
